# baseline (speedup 1.0000x reference)
import functools

import jax
import jax.numpy as jnp
from jax import lax
from jax.experimental import pallas as pl
from jax.experimental.pallas import tpu as pltpu
from jax.experimental.pallas import tpu_sc as plsc

N_V = 100000
M_TETS = 3200000
KT = M_TETS // 128

B = 512
NBLK = M_TETS // B
NW = 32
ITERS_W = (NBLK + NW - 1) // NW
FB = 4 * B
GPB = B // 16
RPB = FB // 128

T_PAD = 100352
STRIPE = T_PAD // 16
HALF = STRIPE // 2

_mesh = plsc.VectorSubcoreMesh(core_axis_name="c", subcore_axis_name="s")
_params = pltpu.CompilerParams(needs_layout_passes=False)


@functools.partial(
    pl.kernel,
    out_type=[
        jax.ShapeDtypeStruct((4 * M_TETS,), jnp.float32),
        jax.ShapeDtypeStruct((4 * M_TETS,), jnp.float32),
        jax.ShapeDtypeStruct((4 * M_TETS,), jnp.float32),
    ],
    mesh=_mesh,
    compiler_params=_params,
    scratch_types=[
        pltpu.VMEM((N_V,), jnp.float32),
        pltpu.VMEM((2 * RPB, 128), jnp.int32),
        pltpu.VMEM((2 * FB,), jnp.float32),
        pltpu.SemaphoreType.DMA,
        pltpu.SemaphoreType.DMA,
    ],
)
def _sc_gather(vpl_hbm, idxr_hbm, xg_hbm, yg_hbm, zg_hbm,
               big_v, idx_v, buf_v, sem_in, sem_out):
    cid = lax.axis_index("c")
    sid = lax.axis_index("s")
    wid = sid * 2 + cid

    def blk_of(i):
        return wid + i * NW

    def _in_pair(i):
        blk = blk_of(i)
        par = jnp.bitwise_and(i, 1)
        return (idxr_hbm.at[pl.ds(blk * RPB, RPB)], idx_v.at[pl.ds(par * RPB, RPB)])

    def issue_in(i):
        @pl.when(blk_of(i) < NBLK)
        def _():
            s, d = _in_pair(i)
            pltpu.async_copy(s, d, sem_in)

    def drain_in(i):
        s, d = _in_pair(i)
        pltpu.make_async_copy(s, d, sem_in).wait()

    for p in range(3):
        pltpu.sync_copy(vpl_hbm.at[pl.ds(p * N_V, N_V)], big_v)
        og_hbm = (xg_hbm, yg_hbm, zg_hbm)[p]

        def _out_pairs(i):
            blk = blk_of(i)
            par = jnp.bitwise_and(i, 1)
            return [
                (buf_v.at[pl.ds(par * FB + c * B, B)],
                 og_hbm.at[pl.ds(c * M_TETS + blk * B, B)])
                for c in range(4)
            ]

        def _drain_out(i):
            for s, d in _out_pairs(i):
                pltpu.make_async_copy(s, d, sem_out).wait()

        issue_in(0)

        def _block(i, _):
            blk = blk_of(i)
            par = jnp.bitwise_and(i, 1)
            issue_in(i + 1)

            @pl.when(blk < NBLK)
            def _():
                drain_in(i)

                @pl.when(i >= 2)
                def _():
                    _drain_out(i - 2)

                def _group(g, _):
                    kt = lax.shift_right_logical(g, 3)
                    row = par * RPB + kt * 4
                    col = jnp.bitwise_and(g, 7) * 16
                    for c in range(4):
                        vid = idx_v[row + c, pl.ds(col, 16)]
                        val = plsc.load_gather(big_v, [vid])
                        buf_v[pl.ds(par * FB + c * B + kt * 128 + col, 16)] = val
                    return 0

                lax.fori_loop(0, GPB, _group, 0)
                for s, d in _out_pairs(i):
                    pltpu.async_copy(s, d, sem_out)
            return 0

        with jax.named_scope("pass_gather"):
            lax.fori_loop(0, ITERS_W, _block, 0)
        for it in (ITERS_W - 3, ITERS_W - 2, ITERS_W - 1):
            @pl.when((blk_of(it) < NBLK) & (blk_of(it + 2) >= NBLK))
            def _():
                _drain_out(it)


@functools.partial(
    pl.kernel,
    out_type=[
        jax.ShapeDtypeStruct((2 * T_PAD,), jnp.float32),
    ],
    mesh=_mesh,
    compiler_params=_params,
    scratch_types=[
        pltpu.VMEM((T_PAD,), jnp.float32),
        pltpu.VMEM((2 * RPB, 128), jnp.int32),
        pltpu.VMEM((2 * B,), jnp.float32),
        pltpu.VMEM((HALF,), jnp.float32),
        pltpu.VMEM((HALF,), jnp.float32),
        pltpu.VMEM_SHARED((16 * HALF,), jnp.float32),
        pltpu.SemaphoreType.DMA,
    ],
)
def _sc_scatter(idxr_hbm, dens_hbm, part_hbm,
                big_v, idx_v, dens_v, acc_v, tin_v, stage, sem_in):
    cid = lax.axis_index("c")
    sid = lax.axis_index("s")
    wid = sid * 2 + cid

    def blk_of(i):
        return wid + i * NW

    def _in_pairs(i):
        blk = blk_of(i)
        par = jnp.bitwise_and(i, 1)
        return [
            (idxr_hbm.at[pl.ds(blk * RPB, RPB)], idx_v.at[pl.ds(par * RPB, RPB)]),
            (dens_hbm.at[pl.ds(blk * B, B)], dens_v.at[pl.ds(par * B, B)]),
        ]

    def issue_in(i):
        @pl.when(blk_of(i) < NBLK)
        def _():
            for s, d in _in_pairs(i):
                pltpu.async_copy(s, d, sem_in)

    def drain_in(i):
        for s, d in _in_pairs(i):
            pltpu.make_async_copy(s, d, sem_in).wait()

    zero16 = jnp.zeros((16,), jnp.float32)

    def _init(i, _):
        big_v[pl.ds(i * 16, 16)] = zero16
        return 0
    lax.fori_loop(0, T_PAD // 16, _init, 0)

    issue_in(0)

    def _sblock(i, _):
        blk = blk_of(i)
        par = jnp.bitwise_and(i, 1)
        issue_in(i + 1)

        @pl.when(blk < NBLK)
        def _():
            drain_in(i)

            def _group(g, _):
                d16 = dens_v[pl.ds(par * B + g * 16, 16)]
                row = par * RPB + lax.shift_right_logical(g, 3) * 4
                col = jnp.bitwise_and(g, 7) * 16
                for c in range(4):
                    idx16 = idx_v[row + c, pl.ds(col, 16)]
                    got = plsc.load_gather(big_v, [idx16])

                    def cond(mask):
                        return jnp.any(mask)

                    def body(mask):
                        plsc.store_scatter(big_v, [idx16], d16, mask=mask)
                        now = plsc.load_gather(big_v, [idx16])
                        return d16 > now

                    lax.while_loop(cond, body, d16 > got)
                return 0

            lax.fori_loop(0, GPB, _group, 0)
        return 0

    with jax.named_scope("pass_scatter"):
        lax.fori_loop(0, ITERS_W, _sblock, 0)

    for h in range(2):
        def _zacc(v, _):
            acc_v[pl.ds(v * 16, 16)] = zero16
            return 0
        lax.fori_loop(0, HALF // 16, _zacc, 0)

        def _round(r, _):
            dst = jnp.bitwise_and(sid + r, 15)
            pltpu.sync_copy(
                big_v.at[pl.ds(dst * STRIPE + h * HALF, HALF)],
                stage.at[pl.ds(dst * HALF, HALF)],
            )
            plsc.subcore_barrier()
            pltpu.sync_copy(stage.at[pl.ds(sid * HALF, HALF)], tin_v)

            def _red_v(v, _):
                sl = pl.ds(v * 16, 16)
                acc_v[sl] = jnp.maximum(acc_v[sl], tin_v[sl])
                return 0
            lax.fori_loop(0, HALF // 16, _red_v, 0)
            plsc.subcore_barrier()
            return 0

        with jax.named_scope("reduce"):
            lax.fori_loop(0, 16, _round, 0)
        pltpu.sync_copy(
            acc_v, part_hbm.at[pl.ds(cid * T_PAD + sid * STRIPE + h * HALF, HALF)]
        )


def _dense(x0, x1, x2, x3, y0, y1, y2, y3, z0, z1, z2, z3, d_ref,
           vol_ref, al_ref):
    xs = [x0[...], x1[...], x2[...], x3[...]]
    ys = [y0[...], y1[...], y2[...], y3[...]]
    zs = [z0[...], z1[...], z2[...], z3[...]]
    e1 = (xs[1] - xs[0], ys[1] - ys[0], zs[1] - zs[0])
    e2 = (xs[2] - xs[0], ys[2] - ys[0], zs[2] - zs[0])
    e3 = (xs[3] - xs[0], ys[3] - ys[0], zs[3] - zs[0])
    det = (
        e1[0] * (e2[1] * e3[2] - e2[2] * e3[1])
        - e1[1] * (e2[0] * e3[2] - e2[2] * e3[0])
        + e1[2] * (e2[0] * e3[1] - e2[1] * e3[0])
    )
    vol_ref[...] = jnp.abs(det) * (1.0 / 6.0)

    def n2(v):
        return v[0] * v[0] + v[1] * v[1] + v[2] * v[2]
    e12 = tuple(e2[k] - e1[k] for k in range(3))
    e13 = tuple(e3[k] - e1[k] for k in range(3))
    e23 = tuple(e3[k] - e2[k] for k in range(3))
    d2 = jnp.minimum(
        jnp.minimum(jnp.minimum(n2(e1), n2(e2)), jnp.minimum(n2(e3), n2(e12))),
        jnp.minimum(n2(e13), n2(e23)),
    )
    el = jnp.sqrt(d2)
    al_ref[...] = 1.0 - jnp.exp(-d_ref[...] * el)


def _combine(a_ref, o_ref):
    o_ref[...] = jnp.maximum(a_ref[0], a_ref[1])


TBR = 8


def kernel(vertices, indices, tet_density):
    vpl = vertices.T.reshape(-1)
    idxr = indices.reshape(KT, 128, 4).transpose(0, 2, 1).reshape(4 * KT, 128)

    xg, yg, zg = _sc_gather(vpl, idxr)
    [part] = _sc_scatter(idxr, tet_density)

    cspecs = [
        pl.BlockSpec((TBR, 128), (lambda i, c=c: (c * (KT // TBR) + i, 0)))
        for c in range(4)
    ] * 3
    rspec = pl.BlockSpec((TBR, 128), lambda i: (i, 0))
    ins = []
    for g in (xg, yg, zg):
        g2 = g.reshape(4 * KT, 128)
        ins.extend([g2, g2, g2, g2])
    vol2, al2 = pl.pallas_call(
        _dense,
        grid=(KT // TBR,),
        in_specs=cspecs + [rspec],
        out_specs=[rspec, rspec],
        out_shape=[
            jax.ShapeDtypeStruct((KT, 128), jnp.float32),
            jax.ShapeDtypeStruct((KT, 128), jnp.float32),
        ],
    )(*ins, tet_density.reshape(KT, 128))

    part3 = part.reshape(2, 8, T_PAD // 8)
    vd = pl.pallas_call(
        _combine,
        out_shape=jax.ShapeDtypeStruct((8, T_PAD // 8), jnp.float32),
    )(part3)
    vertex_density = vd.reshape(T_PAD)[:N_V]
    return (vol2.reshape(M_TETS), al2.reshape(M_TETS), vertex_density)


# speedup vs baseline: 1.6572x; 1.6572x over previous
import functools

import jax
import jax.numpy as jnp
from jax import lax
from jax.experimental import pallas as pl
from jax.experimental.pallas import tpu as pltpu
from jax.experimental.pallas import tpu_sc as plsc

N_V = 100000
M_TETS = 3200000
KT = M_TETS // 128

B = 512
NBLK = M_TETS // B
NW = 32
ITERS_W = (NBLK + NW - 1) // NW
FB = 4 * B
GPB = B // 16
RPB = FB // 128

T_PAD = 100352
STRIPE = T_PAD // 16
HALF = STRIPE // 2

_mesh = plsc.VectorSubcoreMesh(core_axis_name="c", subcore_axis_name="s")
_params = pltpu.CompilerParams(needs_layout_passes=False)


@functools.partial(
    pl.kernel,
    out_type=[
        jax.ShapeDtypeStruct((4 * M_TETS,), jnp.float32),
        jax.ShapeDtypeStruct((4 * M_TETS,), jnp.float32),
        jax.ShapeDtypeStruct((4 * M_TETS,), jnp.float32),
    ],
    mesh=_mesh,
    compiler_params=_params,
    scratch_types=[
        pltpu.VMEM((N_V,), jnp.float32),
        pltpu.VMEM((2 * RPB, 128), jnp.int32),
        pltpu.VMEM((2 * FB,), jnp.float32),
        pltpu.SemaphoreType.DMA,
        pltpu.SemaphoreType.DMA,
    ],
)
def _sc_gather(vpl_hbm, idxr_hbm, xg_hbm, yg_hbm, zg_hbm,
               big_v, idx_v, buf_v, sem_in, sem_out):
    cid = lax.axis_index("c")
    sid = lax.axis_index("s")
    wid = sid * 2 + cid

    def blk_of(i):
        return wid + i * NW

    def _in_pair(i):
        blk = blk_of(i)
        par = jnp.bitwise_and(i, 1)
        return (idxr_hbm.at[pl.ds(blk * RPB, RPB)], idx_v.at[pl.ds(par * RPB, RPB)])

    def issue_in(i):
        @pl.when(blk_of(i) < NBLK)
        def _():
            s, d = _in_pair(i)
            pltpu.async_copy(s, d, sem_in)

    def drain_in(i):
        s, d = _in_pair(i)
        pltpu.make_async_copy(s, d, sem_in).wait()

    for p in range(3):
        pltpu.sync_copy(vpl_hbm.at[pl.ds(p * N_V, N_V)], big_v)
        og_hbm = (xg_hbm, yg_hbm, zg_hbm)[p]

        def _out_pairs(i):
            blk = blk_of(i)
            par = jnp.bitwise_and(i, 1)
            return [
                (buf_v.at[pl.ds(par * FB + c * B, B)],
                 og_hbm.at[pl.ds(c * M_TETS + blk * B, B)])
                for c in range(4)
            ]

        def _drain_out(i):
            for s, d in _out_pairs(i):
                pltpu.make_async_copy(s, d, sem_out).wait()

        issue_in(0)

        def _block(i, _):
            blk = blk_of(i)
            par = jnp.bitwise_and(i, 1)
            issue_in(i + 1)

            @pl.when(blk < NBLK)
            def _():
                drain_in(i)

                @pl.when(i >= 2)
                def _():
                    _drain_out(i - 2)

                def _group(g, _):
                    kt = lax.shift_right_logical(g, 3)
                    row = par * RPB + kt * 4
                    col = jnp.bitwise_and(g, 7) * 16
                    for c in range(4):
                        vid = idx_v[row + c, pl.ds(col, 16)]
                        val = plsc.load_gather(big_v, [vid])
                        buf_v[pl.ds(par * FB + c * B + kt * 128 + col, 16)] = val
                    return 0

                lax.fori_loop(0, GPB, _group, 0)
                for s, d in _out_pairs(i):
                    pltpu.async_copy(s, d, sem_out)
            return 0

        with jax.named_scope("pass_gather"):
            lax.fori_loop(0, ITERS_W, _block, 0)
        for it in (ITERS_W - 3, ITERS_W - 2, ITERS_W - 1):
            @pl.when((blk_of(it) < NBLK) & (blk_of(it + 2) >= NBLK))
            def _():
                _drain_out(it)


@functools.partial(
    pl.kernel,
    out_type=[
        jax.ShapeDtypeStruct((2 * T_PAD,), jnp.float32),
    ],
    mesh=_mesh,
    compiler_params=_params,
    scratch_types=[
        pltpu.VMEM((T_PAD,), jnp.float32),
        pltpu.VMEM((2 * RPB, 128), jnp.int32),
        pltpu.VMEM((2 * B,), jnp.float32),
        pltpu.VMEM((HALF,), jnp.float32),
        pltpu.VMEM((HALF,), jnp.float32),
        pltpu.VMEM_SHARED((16 * HALF,), jnp.float32),
        pltpu.SemaphoreType.DMA,
    ],
)
def _sc_scatter(idxr_hbm, dens_hbm, part_hbm,
                big_v, idx_v, dens_v, acc_v, tin_v, stage, sem_in):
    cid = lax.axis_index("c")
    sid = lax.axis_index("s")
    wid = sid * 2 + cid

    def blk_of(i):
        return wid + i * NW

    def _in_pairs(i):
        blk = blk_of(i)
        par = jnp.bitwise_and(i, 1)
        return [
            (idxr_hbm.at[pl.ds(blk * RPB, RPB)], idx_v.at[pl.ds(par * RPB, RPB)]),
            (dens_hbm.at[pl.ds(blk * B, B)], dens_v.at[pl.ds(par * B, B)]),
        ]

    def issue_in(i):
        @pl.when(blk_of(i) < NBLK)
        def _():
            for s, d in _in_pairs(i):
                pltpu.async_copy(s, d, sem_in)

    def drain_in(i):
        for s, d in _in_pairs(i):
            pltpu.make_async_copy(s, d, sem_in).wait()

    zero16 = jnp.zeros((16,), jnp.float32)

    def _init(i, _):
        big_v[pl.ds(i * 16, 16)] = zero16
        return 0
    lax.fori_loop(0, T_PAD // 16, _init, 0)

    issue_in(0)

    def _sblock(i, _):
        blk = blk_of(i)
        par = jnp.bitwise_and(i, 1)
        issue_in(i + 1)

        @pl.when(blk < NBLK)
        def _():
            drain_in(i)

            def _group(g, _):
                d16 = dens_v[pl.ds(par * B + g * 16, 16)]
                row = par * RPB + lax.shift_right_logical(g, 3) * 4
                col = jnp.bitwise_and(g, 7) * 16
                for c in range(4):
                    idx16 = idx_v[row + c, pl.ds(col, 16)]
                    got = plsc.load_gather(big_v, [idx16])

                    def cond(mask):
                        return jnp.any(mask)

                    def body(mask):
                        plsc.store_scatter(big_v, [idx16], d16, mask=mask)
                        now = plsc.load_gather(big_v, [idx16])
                        return d16 > now

                    lax.while_loop(cond, body, d16 > got)
                return 0

            lax.fori_loop(0, GPB, _group, 0)
        return 0

    with jax.named_scope("pass_scatter"):
        lax.fori_loop(0, ITERS_W, _sblock, 0)

    for h in range(2):
        def _zacc(v, _):
            acc_v[pl.ds(v * 16, 16)] = zero16
            return 0
        lax.fori_loop(0, HALF // 16, _zacc, 0)

        def _round(r, _):
            dst = jnp.bitwise_and(sid + r, 15)
            pltpu.sync_copy(
                big_v.at[pl.ds(dst * STRIPE + h * HALF, HALF)],
                stage.at[pl.ds(dst * HALF, HALF)],
            )
            plsc.subcore_barrier()
            pltpu.sync_copy(stage.at[pl.ds(sid * HALF, HALF)], tin_v)

            def _red_v(v, _):
                sl = pl.ds(v * 16, 16)
                acc_v[sl] = jnp.maximum(acc_v[sl], tin_v[sl])
                return 0
            lax.fori_loop(0, HALF // 16, _red_v, 0)
            plsc.subcore_barrier()
            return 0

        with jax.named_scope("reduce"):
            lax.fori_loop(0, 16, _round, 0)
        pltpu.sync_copy(
            acc_v, part_hbm.at[pl.ds(cid * T_PAD + sid * STRIPE + h * HALF, HALF)]
        )


def _dense(x0, x1, x2, x3, y0, y1, y2, y3, z0, z1, z2, z3, d_ref,
           vol_ref, al_ref):
    xs = [x0[...], x1[...], x2[...], x3[...]]
    ys = [y0[...], y1[...], y2[...], y3[...]]
    zs = [z0[...], z1[...], z2[...], z3[...]]
    e1 = (xs[1] - xs[0], ys[1] - ys[0], zs[1] - zs[0])
    e2 = (xs[2] - xs[0], ys[2] - ys[0], zs[2] - zs[0])
    e3 = (xs[3] - xs[0], ys[3] - ys[0], zs[3] - zs[0])
    det = (
        e1[0] * (e2[1] * e3[2] - e2[2] * e3[1])
        - e1[1] * (e2[0] * e3[2] - e2[2] * e3[0])
        + e1[2] * (e2[0] * e3[1] - e2[1] * e3[0])
    )
    vol_ref[...] = jnp.abs(det) * (1.0 / 6.0)

    def n2(v):
        return v[0] * v[0] + v[1] * v[1] + v[2] * v[2]
    e12 = tuple(e2[k] - e1[k] for k in range(3))
    e13 = tuple(e3[k] - e1[k] for k in range(3))
    e23 = tuple(e3[k] - e2[k] for k in range(3))
    d2 = jnp.minimum(
        jnp.minimum(jnp.minimum(n2(e1), n2(e2)), jnp.minimum(n2(e3), n2(e12))),
        jnp.minimum(n2(e13), n2(e23)),
    )
    el = jnp.sqrt(d2)
    al_ref[...] = 1.0 - jnp.exp(-d_ref[...] * el)


def _combine(a_ref, o_ref):
    o_ref[...] = jnp.maximum(a_ref[0], a_ref[1])


TBR = 200


def kernel(vertices, indices, tet_density):
    vpl = vertices.T.reshape(-1)
    idxr = indices.reshape(KT, 128, 4).transpose(0, 2, 1).reshape(4 * KT, 128)

    xg, yg, zg = _sc_gather(vpl, idxr)
    [part] = _sc_scatter(idxr, tet_density)

    cspecs = [
        pl.BlockSpec((TBR, 128), (lambda i, c=c: (c * (KT // TBR) + i, 0)))
        for c in range(4)
    ] * 3
    rspec = pl.BlockSpec((TBR, 128), lambda i: (i, 0))
    ins = []
    for g in (xg, yg, zg):
        g2 = g.reshape(4 * KT, 128)
        ins.extend([g2, g2, g2, g2])
    vol2, al2 = pl.pallas_call(
        _dense,
        grid=(KT // TBR,),
        in_specs=cspecs + [rspec],
        out_specs=[rspec, rspec],
        out_shape=[
            jax.ShapeDtypeStruct((KT, 128), jnp.float32),
            jax.ShapeDtypeStruct((KT, 128), jnp.float32),
        ],
    )(*ins, tet_density.reshape(KT, 128))

    part3 = part.reshape(2, 8, T_PAD // 8)
    vd = pl.pallas_call(
        _combine,
        out_shape=jax.ShapeDtypeStruct((8, T_PAD // 8), jnp.float32),
    )(part3)
    vertex_density = vd.reshape(T_PAD)[:N_V]
    return (vol2.reshape(M_TETS), al2.reshape(M_TETS), vertex_density)


# speedup vs baseline: 1.6760x; 1.0114x over previous
import functools

import jax
import jax.numpy as jnp
from jax import lax
from jax.experimental import pallas as pl
from jax.experimental.pallas import tpu as pltpu
from jax.experimental.pallas import tpu_sc as plsc

N_V = 100000
M_TETS = 3200000
KT = M_TETS // 128

B = 512
NBLK = M_TETS // B
NW = 32
ITERS_W = (NBLK + NW - 1) // NW
FB = 4 * B
GPB = B // 16
RPB = FB // 128

T_PAD = 100352
STRIPE = T_PAD // 16
HALF = STRIPE // 2

_mesh = plsc.VectorSubcoreMesh(core_axis_name="c", subcore_axis_name="s")


def _rsqrt(x):
    i = plsc.bitcast(x, jnp.int32)
    i = jnp.int32(0x5F3759DF) - lax.shift_right_arithmetic(i, 1)
    y = plsc.bitcast(i, jnp.float32)
    for _ in range(3):
        y = y * (1.5 - 0.5 * x * y * y)
    return y


@functools.partial(
    pl.kernel,
    out_type=[
        jax.ShapeDtypeStruct((M_TETS,), jnp.float32),
        jax.ShapeDtypeStruct((M_TETS,), jnp.float32),
        jax.ShapeDtypeStruct((2 * T_PAD,), jnp.float32),
        jax.ShapeDtypeStruct((4 * M_TETS,), jnp.float32),
        jax.ShapeDtypeStruct((4 * M_TETS,), jnp.float32),
    ],
    mesh=_mesh,
    compiler_params=pltpu.CompilerParams(needs_layout_passes=False),
    scratch_types=[
        pltpu.VMEM((T_PAD,), jnp.float32),
        pltpu.VMEM((2 * RPB, 128), jnp.int32),
        pltpu.VMEM((2 * FB,), jnp.float32),
        pltpu.VMEM((2 * FB,), jnp.float32),
        pltpu.VMEM((2 * B,), jnp.float32),
        pltpu.VMEM((2 * B,), jnp.float32),
        pltpu.VMEM((2 * B,), jnp.float32),
        pltpu.VMEM((HALF,), jnp.float32),
        pltpu.VMEM((HALF,), jnp.float32),
        pltpu.VMEM_SHARED((16 * HALF,), jnp.float32),
        pltpu.SemaphoreType.DMA,
        pltpu.SemaphoreType.DMA,
    ],
)
def _sc_tets(vpl_hbm, idxr_hbm, dens_hbm, vol_hbm, alpha_hbm, part_hbm,
             xg_hbm, yg_hbm,
             big_v, idx_v, bufx_v, bufy_v, dens_v, vol_v, al_v,
             acc_v, tin_v, stage, sem_in, sem_out):
    cid = lax.axis_index("c")
    sid = lax.axis_index("s")
    wid = sid * 2 + cid

    def blk_of(i):
        return wid + i * NW

    def _in_pairs(i, want_dens, want_xy):
        blk = blk_of(i)
        par = jnp.bitwise_and(i, 1)
        pairs = [(idxr_hbm.at[pl.ds(blk * RPB, RPB)], idx_v.at[pl.ds(par * RPB, RPB)])]
        if want_dens:
            pairs.append((dens_hbm.at[pl.ds(blk * B, B)], dens_v.at[pl.ds(par * B, B)]))
        if want_xy:
            pairs.append((xg_hbm.at[pl.ds(blk * FB, FB)], bufx_v.at[pl.ds(par * FB, FB)]))
            pairs.append((yg_hbm.at[pl.ds(blk * FB, FB)], bufy_v.at[pl.ds(par * FB, FB)]))
        return pairs

    def issue_in(i, want_dens=False, want_xy=False):
        @pl.when(blk_of(i) < NBLK)
        def _():
            for s, d in _in_pairs(i, want_dens, want_xy):
                pltpu.async_copy(s, d, sem_in)

    def drain_in(i, want_dens=False, want_xy=False):
        for s, d in _in_pairs(i, want_dens, want_xy):
            pltpu.make_async_copy(s, d, sem_in).wait()

    for p in range(2):
        pltpu.sync_copy(vpl_hbm.at[pl.ds(p * N_V, N_V)], big_v.at[pl.ds(0, N_V)])
        og_hbm = (xg_hbm, yg_hbm)[p]
        ob_v = (bufx_v, bufy_v)[p]

        def _out_pair(i):
            blk = blk_of(i)
            par = jnp.bitwise_and(i, 1)
            return ob_v.at[pl.ds(par * FB, FB)], og_hbm.at[pl.ds(blk * FB, FB)]

        def _drain_out(i):
            s, d = _out_pair(i)
            pltpu.make_async_copy(s, d, sem_out).wait()

        issue_in(0)

        def _block(i, _):
            blk = blk_of(i)
            par = jnp.bitwise_and(i, 1)
            issue_in(i + 1)

            @pl.when(blk < NBLK)
            def _():
                drain_in(i)

                @pl.when(i >= 2)
                def _():
                    _drain_out(i - 2)

                def _group(g, _):
                    row = par * RPB + lax.shift_right_logical(g, 3) * 4
                    col = jnp.bitwise_and(g, 7) * 16
                    pbase = par * FB + lax.shift_right_logical(g, 3) * 512 + jnp.bitwise_and(g, 7) * 16
                    for c in range(4):
                        vid = idx_v[row + c, pl.ds(col, 16)]
                        val = plsc.load_gather(big_v, [vid])
                        ob_v[pl.ds(pbase + c * 128, 16)] = val
                    return 0

                lax.fori_loop(0, GPB, _group, 0)
                s, d = _out_pair(i)
                pltpu.async_copy(s, d, sem_out)
            return 0

        with jax.named_scope("pass_gather"):
            lax.fori_loop(0, ITERS_W, _block, 0)
        for it in (ITERS_W - 3, ITERS_W - 2, ITERS_W - 1):
            @pl.when((blk_of(it) < NBLK) & (blk_of(it + 2) >= NBLK))
            def _():
                _drain_out(it)

    pltpu.sync_copy(vpl_hbm.at[pl.ds(2 * N_V, N_V)], big_v.at[pl.ds(0, N_V)])

    def _zout_pairs(i):
        blk = blk_of(i)
        par = jnp.bitwise_and(i, 1)
        return [
            (vol_v.at[pl.ds(par * B, B)], vol_hbm.at[pl.ds(blk * B, B)]),
            (al_v.at[pl.ds(par * B, B)], alpha_hbm.at[pl.ds(blk * B, B)]),
        ]

    def _zdrain_out(i):
        for s, d in _zout_pairs(i):
            pltpu.make_async_copy(s, d, sem_out).wait()

    issue_in(0, want_dens=True, want_xy=True)

    def _zblock(i, _):
        blk = blk_of(i)
        par = jnp.bitwise_and(i, 1)
        issue_in(i + 1, want_dens=True, want_xy=True)

        @pl.when(blk < NBLK)
        def _():
            drain_in(i, want_dens=True, want_xy=True)

            @pl.when(i >= 2)
            def _():
                _zdrain_out(i - 2)

            def _group(g, _):
                xs, ys, zs = [], [], []
                row = par * RPB + lax.shift_right_logical(g, 3) * 4
                col = jnp.bitwise_and(g, 7) * 16
                pbase = par * FB + lax.shift_right_logical(g, 3) * 512 + jnp.bitwise_and(g, 7) * 16
                for c in range(4):
                    sl = pl.ds(pbase + c * 128, 16)
                    xs.append(bufx_v[sl])
                    ys.append(bufy_v[sl])
                    vid = idx_v[row + c, pl.ds(col, 16)]
                    zs.append(plsc.load_gather(big_v, [vid]))
                e1 = (xs[1] - xs[0], ys[1] - ys[0], zs[1] - zs[0])
                e2 = (xs[2] - xs[0], ys[2] - ys[0], zs[2] - zs[0])
                e3 = (xs[3] - xs[0], ys[3] - ys[0], zs[3] - zs[0])
                det = (
                    e1[0] * (e2[1] * e3[2] - e2[2] * e3[1])
                    - e1[1] * (e2[0] * e3[2] - e2[2] * e3[0])
                    + e1[2] * (e2[0] * e3[1] - e2[1] * e3[0])
                )
                vol_v[pl.ds(par * B + g * 16, 16)] = jnp.abs(det) * (1.0 / 6.0)

                def n2(v):
                    return v[0] * v[0] + v[1] * v[1] + v[2] * v[2]
                e12 = tuple(e2[k] - e1[k] for k in range(3))
                e13 = tuple(e3[k] - e1[k] for k in range(3))
                e23 = tuple(e3[k] - e2[k] for k in range(3))
                d2 = jnp.minimum(
                    jnp.minimum(jnp.minimum(n2(e1), n2(e2)),
                                jnp.minimum(n2(e3), n2(e12))),
                    jnp.minimum(n2(e13), n2(e23)),
                )
                d2 = jnp.maximum(d2, 1e-24)
                el = d2 * _rsqrt(d2)
                d16 = dens_v[pl.ds(par * B + g * 16, 16)]
                al_v[pl.ds(par * B + g * 16, 16)] = 1.0 - jnp.exp(-d16 * el)
                return 0

            lax.fori_loop(0, GPB, _group, 0)
            for s, d in _zout_pairs(i):
                pltpu.async_copy(s, d, sem_out)
        return 0

    with jax.named_scope("pass_z"):
        lax.fori_loop(0, ITERS_W, _zblock, 0)
    for it in (ITERS_W - 3, ITERS_W - 2, ITERS_W - 1):
        @pl.when((blk_of(it) < NBLK) & (blk_of(it + 2) >= NBLK))
        def _():
            _zdrain_out(it)

    zero16 = jnp.zeros((16,), jnp.float32)

    def _init(i, _):
        big_v[pl.ds(i * 16, 16)] = zero16
        return 0
    lax.fori_loop(0, T_PAD // 16, _init, 0)

    issue_in(0, want_dens=True)

    def _sblock(i, _):
        blk = blk_of(i)
        par = jnp.bitwise_and(i, 1)
        issue_in(i + 1, want_dens=True)

        @pl.when(blk < NBLK)
        def _():
            drain_in(i, want_dens=True)

            def _group(g, _):
                d16 = dens_v[pl.ds(par * B + g * 16, 16)]
                row = par * RPB + lax.shift_right_logical(g, 3) * 4
                col = jnp.bitwise_and(g, 7) * 16
                for c in range(4):
                    idx16 = idx_v[row + c, pl.ds(col, 16)]
                    got = plsc.load_gather(big_v, [idx16])

                    def cond(mask):
                        return jnp.any(mask)

                    def body(mask):
                        plsc.store_scatter(big_v, [idx16], d16, mask=mask)
                        now = plsc.load_gather(big_v, [idx16])
                        return d16 > now

                    lax.while_loop(cond, body, d16 > got)
                return 0

            lax.fori_loop(0, GPB, _group, 0)
        return 0

    with jax.named_scope("pass_scatter"):
        lax.fori_loop(0, ITERS_W, _sblock, 0)

    for h in range(2):
        def _zacc(v, _):
            acc_v[pl.ds(v * 16, 16)] = zero16
            return 0
        lax.fori_loop(0, HALF // 16, _zacc, 0)

        def _round(r, _):
            dst = jnp.bitwise_and(sid + r, 15)
            pltpu.sync_copy(
                big_v.at[pl.ds(dst * STRIPE + h * HALF, HALF)],
                stage.at[pl.ds(dst * HALF, HALF)],
            )
            plsc.subcore_barrier()
            pltpu.sync_copy(stage.at[pl.ds(sid * HALF, HALF)], tin_v)

            def _red_v(v, _):
                sl = pl.ds(v * 16, 16)
                acc_v[sl] = jnp.maximum(acc_v[sl], tin_v[sl])
                return 0
            lax.fori_loop(0, HALF // 16, _red_v, 0)
            plsc.subcore_barrier()
            return 0

        with jax.named_scope("reduce"):
            lax.fori_loop(0, 16, _round, 0)
        pltpu.sync_copy(
            acc_v, part_hbm.at[pl.ds(cid * T_PAD + sid * STRIPE + h * HALF, HALF)]
        )


def _combine(a_ref, o_ref):
    o_ref[...] = jnp.maximum(a_ref[0], a_ref[1])


def kernel(vertices, indices, tet_density):
    vpl = vertices.T.reshape(-1)
    idxr = indices.reshape(KT, 128, 4).transpose(0, 2, 1).reshape(4 * KT, 128)
    vol, alpha, part, _, _ = _sc_tets(vpl, idxr, tet_density)

    part3 = part.reshape(2, 8, T_PAD // 8)
    vd = pl.pallas_call(
        _combine,
        out_shape=jax.ShapeDtypeStruct((8, T_PAD // 8), jnp.float32),
    )(part3)
    vertex_density = vd.reshape(T_PAD)[:N_V]
    return (vol, alpha, vertex_density)
